# baseline (speedup 1.0000x reference)
import functools

import jax
import jax.numpy as jnp
from jax import lax
from jax.experimental import pallas as pl
from jax.experimental.pallas import tpu as pltpu
from jax.experimental.pallas import tpu_sc as plsc

_BQ = 512


def _warp_kernel(x1_ref, x2_ref, f1_ref, out_ref, *, n1):
    x1 = x1_ref[0]
    f1 = f1_ref[0]
    qs = x2_ref[0]
    db = x1 + f1
    q = qs.T

    d = None
    for c in range(3):
        diff = q[:, c:c + 1] - db[c:c + 1, :]
        d = diff * diff if d is None else d + diff * diff

    iota = jax.lax.broadcasted_iota(jnp.int32, d.shape, 1)
    inf = jnp.float32(jnp.inf)

    invs = []
    dcur = d
    W = None
    for k in range(3):
        m = jnp.min(dcur, axis=1, keepdims=True)
        idx = jnp.min(jnp.where(dcur <= m, iota, n1), axis=1,
                      keepdims=True)
        oh = iota == idx
        inv = 1.0 / jnp.maximum(jnp.sqrt(m), 1e-10)
        invs.append(inv)
        W = jnp.where(oh, inv, 0.0) if k == 0 else jnp.where(oh, inv, W)
        if k < 2:
            dcur = jnp.where(oh, inf, dcur)

    rnorm = 1.0 / (invs[0] + invs[1] + invs[2])

    s_t = jax.lax.dot_general(f1, W, (((1,), (1,)), ((), ())),
                              preferred_element_type=jnp.float32)
    out_ref[0] = qs - s_t * rnorm.T


_SC_NW = 32


def _rsqrt_newton(x):
    xh = x * 0.5
    i = lax.bitcast_convert_type(x, jnp.int32)
    i = jnp.int32(0x5F3759DF) - jnp.right_shift(i, 1)
    y = lax.bitcast_convert_type(i, jnp.float32)
    for _ in range(4):
        y = y * (1.5 - xh * y * y)
    return y


def _permute(v, perm):
    dnums = lax.GatherDimensionNumbers(offset_dims=(),
                                       collapsed_slice_dims=(0,),
                                       start_index_map=(0,))
    return lax.gather(v, perm[:, None], dnums, slice_sizes=(1,),
                      mode=lax.GatherScatterMode.PROMISE_IN_BOUNDS)


def _allmin(v, iota16):
    for sh in (8, 4, 2, 1):
        v = jnp.minimum(v, _permute(v, jnp.bitwise_xor(iota16, sh)))
    return v


def _sc_body(x1_hbm, x2_hbm, f1_hbm, out_hbm, db_v, f1_v, f1x_v, f1y_v,
             f1z_v, q_v, out_v, mk_v, ik_v, *, nq, n1, wpb):
    wid = lax.axis_index("s") * 2 + lax.axis_index("c")
    b = wid // wpb
    qoff = (wid % wpb) * nq
    nch = n1 // 16

    pltpu.sync_copy(x1_hbm.at[b], db_v)
    pltpu.sync_copy(f1_hbm.at[b], f1_v)
    pltpu.sync_copy(x2_hbm.at[b, :, pl.ds(qoff, nq)], q_v)

    f1rows = (f1x_v, f1y_v, f1z_v)

    def add_body(j, _):
        s = pl.ds(j * 16, 16)
        for c in range(3):
            f = f1_v[c, s]
            f1rows[c][s] = f
            db_v[c, s] = db_v[c, s] + f
        return 0
    lax.fori_loop(0, nch, add_body, 0)

    iota16 = lax.broadcasted_iota(jnp.int32, (16,), 0)
    zi = iota16 * 0
    zf = zi.astype(jnp.float32)
    bigf = jnp.float32(3e38)
    bigi = jnp.int32(2 ** 30)
    vbigf = zf + bigf
    vbigi = zi + bigi

    def grp_body(e, _):
        s = pl.ds(e * 16, 16)
        qxv = q_v[0, s]
        qyv = q_v[1, s]
        qzv = q_v[2, s]
        mg = [vbigf] * 3
        ig = [vbigi] * 3
        for lane in range(16):
            lperm = zi + lane
            qx = _permute(qxv, lperm)
            qy = _permute(qyv, lperm)
            qz = _permute(qzv, lperm)

            def chunk(j, carry):
                m1, m2, m3, i1, i2, i3 = carry
                cs = pl.ds(j * 16, 16)
                dx = db_v[0, cs] - qx
                dy = db_v[1, cs] - qy
                dz = db_v[2, cs] - qz
                v = dx * dx + dy * dy + dz * dz
                jv = j * 16 + iota16
                c1 = v < m1
                c2 = v < m2
                c3 = v < m3
                m3n = jnp.where(c3, jnp.where(c2, m2, v), m3)
                i3n = jnp.where(c3, jnp.where(c2, i2, jv), i3)
                m2n = jnp.where(c2, jnp.where(c1, m1, v), m2)
                i2n = jnp.where(c2, jnp.where(c1, i1, jv), i2)
                m1n = jnp.where(c1, v, m1)
                i1n = jnp.where(c1, jv, i1)
                return (m1n, m2n, m3n, i1n, i2n, i3n)

            init = (vbigf, vbigf, vbigf, vbigi, vbigi, vbigi)
            m1, m2, m3, i1, i2, i3 = lax.fori_loop(0, nch, chunk, init)

            lmask = iota16 == lane
            for k in range(3):
                vm = jnp.minimum(jnp.minimum(m1, m2), m3)
                mk = _allmin(vm, iota16)
                iw = jnp.minimum(jnp.minimum(jnp.where(m1 == mk, i1, vbigi),
                                             jnp.where(m2 == mk, i2, vbigi)),
                                 jnp.where(m3 == mk, i3, vbigi))
                ik = _allmin(iw, iota16)
                mg[k] = jnp.where(lmask, mk, mg[k])
                ig[k] = jnp.where(lmask, ik, ig[k])
                m1 = jnp.where((m1 == mk) & (i1 == ik), bigf, m1)
                m2 = jnp.where((m2 == mk) & (i2 == ik), bigf, m2)
                m3 = jnp.where((m3 == mk) & (i3 == ik), bigf, m3)
        for k in range(3):
            mk_v[k, s] = mg[k]
            ik_v[k, s] = ig[k]
        return 0
    lax.fori_loop(0, nq // 16, grp_body, 0)

    def ep_body(e, _):
        s = pl.ds(e * 16, 16)
        invs = [jnp.minimum(_rsqrt_newton(mk_v[k, s]), jnp.float32(1e10))
                for k in range(3)]
        rn = 1.0 / (invs[0] + invs[1] + invs[2])
        idxs = [ik_v[k, s] for k in range(3)]
        for c in range(3):
            g = [plsc.load_gather(f1rows[c], [idxs[k]]) for k in range(3)]
            out_v[c, s] = q_v[c, s] - (invs[0] * g[0] + invs[1] * g[1]
                                       + invs[2] * g[2]) * rn
        return 0
    lax.fori_loop(0, nq // 16, ep_body, 0)

    pltpu.sync_copy(out_v, out_hbm.at[b, :, pl.ds(qoff, nq)])


def _sc_warp(xyz1, xyz2, flow1, nq_total):
    B, C, N1 = xyz1.shape
    wpb = _SC_NW // B
    nq = nq_total // wpb
    mesh = plsc.VectorSubcoreMesh(core_axis_name="c", subcore_axis_name="s")
    return pl.kernel(
        functools.partial(_sc_body, nq=nq, n1=N1, wpb=wpb),
        out_type=jax.ShapeDtypeStruct((B, C, nq_total), jnp.float32),
        mesh=mesh,
        compiler_params=pltpu.CompilerParams(needs_layout_passes=False),
        scratch_types=[
            pltpu.VMEM((3, N1), jnp.float32),
            pltpu.VMEM((3, N1), jnp.float32),
            pltpu.VMEM((N1,), jnp.float32),
            pltpu.VMEM((N1,), jnp.float32),
            pltpu.VMEM((N1,), jnp.float32),
            pltpu.VMEM((3, nq), jnp.float32),
            pltpu.VMEM((3, nq), jnp.float32),
            pltpu.VMEM((3, nq), jnp.float32),
            pltpu.VMEM((3, nq), jnp.int32),
        ],
    )(xyz1, xyz2, flow1)


@jax.jit
def kernel(xyz1, xyz2, flow1):
    return _sc_warp(xyz1, xyz2, flow1, xyz2.shape[2])


@jax.jit
def _tc_kernel(xyz1, xyz2, flow1):
    B, C, N1 = xyz1.shape
    N2 = xyz2.shape[2]

    return pl.pallas_call(
        functools.partial(_warp_kernel, n1=N1),
        grid=(B, N2 // _BQ),
        in_specs=[
            pl.BlockSpec((1, C, N1), lambda b, i: (b, 0, 0)),
            pl.BlockSpec((1, C, _BQ), lambda b, i: (b, 0, i)),
            pl.BlockSpec((1, C, N1), lambda b, i: (b, 0, 0)),
        ],
        out_specs=pl.BlockSpec((1, C, _BQ), lambda b, i: (b, 0, i)),
        out_shape=jax.ShapeDtypeStruct((B, C, N2), jnp.float32),
        compiler_params=pltpu.CompilerParams(
            dimension_semantics=("parallel", "parallel")),
    )(xyz1, xyz2, flow1)


# speedup vs baseline: 3.1336x; 3.1336x over previous
import functools

import jax
import jax.numpy as jnp
from jax import lax
from jax.experimental import pallas as pl
from jax.experimental.pallas import tpu as pltpu
from jax.experimental.pallas import tpu_sc as plsc

_BQ = 512


def _warp_kernel(x1_ref, x2_ref, f1_ref, out_ref, *, n1):
    x1 = x1_ref[0]
    f1 = f1_ref[0]
    qs = x2_ref[0]
    db = x1 + f1
    q = qs.T

    d = None
    for c in range(3):
        diff = q[:, c:c + 1] - db[c:c + 1, :]
        d = diff * diff if d is None else d + diff * diff

    iota = jax.lax.broadcasted_iota(jnp.int32, d.shape, 1)
    inf = jnp.float32(jnp.inf)

    invs = []
    dcur = d
    W = None
    for k in range(3):
        m = jnp.min(dcur, axis=1, keepdims=True)
        idx = jnp.min(jnp.where(dcur <= m, iota, n1), axis=1,
                      keepdims=True)
        oh = iota == idx
        inv = 1.0 / jnp.maximum(jnp.sqrt(m), 1e-10)
        invs.append(inv)
        W = jnp.where(oh, inv, 0.0) if k == 0 else jnp.where(oh, inv, W)
        if k < 2:
            dcur = jnp.where(oh, inf, dcur)

    rnorm = 1.0 / (invs[0] + invs[1] + invs[2])

    s_t = jax.lax.dot_general(f1, W, (((1,), (1,)), ((), ())),
                              preferred_element_type=jnp.float32)
    out_ref[0] = qs - s_t * rnorm.T


_SC_NW = 32


def _rsqrt_newton(x):
    xh = x * 0.5
    i = lax.bitcast_convert_type(x, jnp.int32)
    i = jnp.int32(0x5F3759DF) - jnp.right_shift(i, 1)
    y = lax.bitcast_convert_type(i, jnp.float32)
    for _ in range(4):
        y = y * (1.5 - xh * y * y)
    return y


def _permute(v, perm):
    dnums = lax.GatherDimensionNumbers(offset_dims=(),
                                       collapsed_slice_dims=(0,),
                                       start_index_map=(0,))
    return lax.gather(v, perm[:, None], dnums, slice_sizes=(1,),
                      mode=lax.GatherScatterMode.PROMISE_IN_BOUNDS)


def _allmin(v, iota16):
    for sh in (8, 4, 2, 1):
        v = jnp.minimum(v, _permute(v, jnp.bitwise_xor(iota16, sh)))
    return v


def _sc_body(x1_hbm, x2_hbm, f1_hbm, out_hbm, db_v, f1_v, f1x_v, f1y_v,
             f1z_v, q_v, out_v, mk_v, ik_v, *, nq, n1, wpb):
    wid = lax.axis_index("s") * 2 + lax.axis_index("c")
    b = wid // wpb
    qoff = (wid % wpb) * nq
    nch = n1 // 16

    pltpu.sync_copy(x1_hbm.at[b], db_v)
    pltpu.sync_copy(f1_hbm.at[b], f1_v)
    pltpu.sync_copy(x2_hbm.at[b, :, pl.ds(qoff, nq)], q_v)

    f1rows = (f1x_v, f1y_v, f1z_v)

    def add_body(j, _):
        s = pl.ds(j * 16, 16)
        for c in range(3):
            f = f1_v[c, s]
            f1rows[c][s] = f
            db_v[c, s] = db_v[c, s] + f
        return 0
    lax.fori_loop(0, nch, add_body, 0)

    iota16 = lax.broadcasted_iota(jnp.int32, (16,), 0)
    zi = iota16 * 0
    zf = zi.astype(jnp.float32)
    bigf = jnp.float32(3e38)
    bigi = jnp.int32(2 ** 30)
    vbigf = zf + bigf
    vbigi = zi + bigi

    def grp_body(e, _):
        s = pl.ds(e * 16, 16)
        qxv = q_v[0, s]
        qyv = q_v[1, s]
        qzv = q_v[2, s]
        mg = [vbigf] * 3
        ig = [vbigi] * 3
        for lane in range(16):
            lperm = zi + lane
            qx = _permute(qxv, lperm)
            qy = _permute(qyv, lperm)
            qz = _permute(qzv, lperm)

            def chunk(j, carry):
                m1, m2, m3, i1, i2, i3 = carry
                cs = pl.ds(j * 16, 16)
                dx = db_v[0, cs] - qx
                dy = db_v[1, cs] - qy
                dz = db_v[2, cs] - qz
                v = dx * dx + dy * dy + dz * dz
                jv = j * 16 + iota16
                c1 = v < m1
                c2 = v < m2
                c3 = v < m3
                m3n = jnp.where(c3, jnp.where(c2, m2, v), m3)
                i3n = jnp.where(c3, jnp.where(c2, i2, jv), i3)
                m2n = jnp.where(c2, jnp.where(c1, m1, v), m2)
                i2n = jnp.where(c2, jnp.where(c1, i1, jv), i2)
                m1n = jnp.where(c1, v, m1)
                i1n = jnp.where(c1, jv, i1)
                return (m1n, m2n, m3n, i1n, i2n, i3n)

            init = (vbigf, vbigf, vbigf, vbigi, vbigi, vbigi)
            m1, m2, m3, i1, i2, i3 = lax.fori_loop(0, nch, chunk, init)

            lmask = iota16 == lane
            for k in range(3):
                vm = jnp.minimum(jnp.minimum(m1, m2), m3)
                mk = _allmin(vm, iota16)
                iw = jnp.minimum(jnp.minimum(jnp.where(m1 == mk, i1, vbigi),
                                             jnp.where(m2 == mk, i2, vbigi)),
                                 jnp.where(m3 == mk, i3, vbigi))
                ik = _allmin(iw, iota16)
                mg[k] = jnp.where(lmask, mk, mg[k])
                ig[k] = jnp.where(lmask, ik, ig[k])
                m1 = jnp.where((m1 == mk) & (i1 == ik), bigf, m1)
                m2 = jnp.where((m2 == mk) & (i2 == ik), bigf, m2)
                m3 = jnp.where((m3 == mk) & (i3 == ik), bigf, m3)
        for k in range(3):
            mk_v[k, s] = mg[k]
            ik_v[k, s] = ig[k]
        return 0
    lax.fori_loop(0, nq // 16, grp_body, 0)

    def ep_body(e, _):
        s = pl.ds(e * 16, 16)
        invs = [jnp.minimum(_rsqrt_newton(mk_v[k, s]), jnp.float32(1e10))
                for k in range(3)]
        rn = 1.0 / (invs[0] + invs[1] + invs[2])
        idxs = [ik_v[k, s] for k in range(3)]
        for c in range(3):
            g = [plsc.load_gather(f1rows[c], [idxs[k]]) for k in range(3)]
            out_v[c, s] = q_v[c, s] - (invs[0] * g[0] + invs[1] * g[1]
                                       + invs[2] * g[2]) * rn
        return 0
    lax.fori_loop(0, nq // 16, ep_body, 0)

    pltpu.sync_copy(out_v, out_hbm.at[b, :, pl.ds(qoff, nq)])


def _sc_warp(xyz1, xyz2, flow1, nq_total):
    B, C, N1 = xyz1.shape
    wpb = _SC_NW // B
    nq = nq_total // wpb
    mesh = plsc.VectorSubcoreMesh(core_axis_name="c", subcore_axis_name="s")
    return pl.kernel(
        functools.partial(_sc_body, nq=nq, n1=N1, wpb=wpb),
        out_type=jax.ShapeDtypeStruct((B, C, nq_total), jnp.float32),
        mesh=mesh,
        compiler_params=pltpu.CompilerParams(needs_layout_passes=False),
        scratch_types=[
            pltpu.VMEM((3, N1), jnp.float32),
            pltpu.VMEM((3, N1), jnp.float32),
            pltpu.VMEM((N1,), jnp.float32),
            pltpu.VMEM((N1,), jnp.float32),
            pltpu.VMEM((N1,), jnp.float32),
            pltpu.VMEM((3, nq), jnp.float32),
            pltpu.VMEM((3, nq), jnp.float32),
            pltpu.VMEM((3, nq), jnp.float32),
            pltpu.VMEM((3, nq), jnp.int32),
        ],
    )(xyz1, xyz2, flow1)


def _tc_warp(xyz1, xyz2_tc, flow1):
    B, C, N1 = xyz1.shape
    n2 = xyz2_tc.shape[2]

    return pl.pallas_call(
        functools.partial(_warp_kernel, n1=N1),
        grid=(B, n2 // _BQ),
        in_specs=[
            pl.BlockSpec((1, C, N1), lambda b, i: (b, 0, 0)),
            pl.BlockSpec((1, C, _BQ), lambda b, i: (b, 0, i)),
            pl.BlockSpec((1, C, N1), lambda b, i: (b, 0, 0)),
        ],
        out_specs=pl.BlockSpec((1, C, _BQ), lambda b, i: (b, 0, i)),
        out_shape=jax.ShapeDtypeStruct((B, C, n2), jnp.float32),
        compiler_params=pltpu.CompilerParams(
            dimension_semantics=("parallel", "parallel")),
    )(xyz1, xyz2_tc, flow1)


_N_SC = 1024


@jax.jit
def kernel(xyz1, xyz2, flow1):
    N2 = xyz2.shape[2]
    ntc = N2 - _N_SC
    out_tc = _tc_warp(xyz1, xyz2[:, :, :ntc], flow1)
    out_sc = _sc_warp(xyz1, xyz2[:, :, ntc:], flow1, _N_SC)
    return jnp.concatenate([out_tc, out_sc], axis=2)
